# baseline (speedup 1.0000x reference)
import functools

import jax
import jax.numpy as jnp
from jax import lax
from jax.experimental import pallas as pl
from jax.experimental.pallas import tpu as pltpu
from jax.experimental.pallas import tpu_sc as plsc

B = 16384
V = 1000000
D = 64
H1 = 384
H2 = 192
NC_OUT = 173
EPS = 1e-5

_NUM_WORKERS = 32
_B_PER_W = B // _NUM_WORKERS


@functools.partial(
    pl.kernel,
    out_type=jax.ShapeDtypeStruct((B, D), jnp.float32),
    mesh=plsc.VectorSubcoreMesh(core_axis_name="c", subcore_axis_name="s"),
    scratch_types=[
        pltpu.VMEM((_B_PER_W,), jnp.int32),
        pltpu.VMEM((_B_PER_W, D), jnp.float32),
        pltpu.SemaphoreType.DMA,
    ],
    compiler_params=pltpu.CompilerParams(use_tc_tiling_on_sc=False),
)
def _sc_gather(table_hbm, idx_hbm, out_hbm, idx_v, rows_v, sem):
    wid = lax.axis_index("s") * 2 + lax.axis_index("c")
    base = wid * _B_PER_W
    pltpu.sync_copy(idx_hbm.at[pl.ds(base, _B_PER_W)], idx_v)
    pltpu.async_copy(table_hbm.at[idx_v], rows_v, sem).wait()
    pltpu.sync_copy(rows_v, out_hbm.at[pl.ds(base, _B_PER_W)])


def _mlp_body(h_ref, w1_ref, b1_ref, w2_ref, b2_ref, wo_ref, bo_ref, out_ref):
    h = h_ref[...]
    z1 = jnp.dot(h, w1_ref[...], preferred_element_type=jnp.float32)
    z1 = z1 + b1_ref[...]
    h1 = 0.5 * z1 * (1.0 + lax.erf(z1 * 0.7071067811865476))
    z2 = jnp.dot(h1, w2_ref[...], preferred_element_type=jnp.float32)
    z2 = z2 + b2_ref[...]
    h2 = 0.5 * z2 * (1.0 + lax.erf(z2 * 0.7071067811865476))
    out = jnp.dot(h2, wo_ref[...], preferred_element_type=jnp.float32)
    out_ref[...] = out + bo_ref[...]


_BLK = 2048


def _mlp(h, w1t, b1f, w2t, b2f, wot, bout):
    grid = (B // _BLK,)
    return pl.pallas_call(
        _mlp_body,
        grid=grid,
        in_specs=[
            pl.BlockSpec((_BLK, D), lambda i: (i, 0)),
            pl.BlockSpec((D, H1), lambda i: (0, 0)),
            pl.BlockSpec((1, H1), lambda i: (0, 0)),
            pl.BlockSpec((H1, H2), lambda i: (0, 0)),
            pl.BlockSpec((1, H2), lambda i: (0, 0)),
            pl.BlockSpec((H2, NC_OUT), lambda i: (0, 0)),
            pl.BlockSpec((1, NC_OUT), lambda i: (0, 0)),
        ],
        out_specs=pl.BlockSpec((_BLK, NC_OUT), lambda i: (i, 0)),
        out_shape=jax.ShapeDtypeStruct((B, NC_OUT), jnp.float32),
    )(h, w1t, b1f, w2t, b2f, wot, bout)


def kernel(x, emb, g0, be0, rm0, rv0, W1, b1, g1, be1, rm1, rv1,
           W2, b2, g2, be2, rm2, rv2, Wout, bout):
    s0 = g0 / jnp.sqrt(rv0 + EPS)
    t0 = be0 - rm0 * s0
    s1 = g1 / jnp.sqrt(rv1 + EPS)
    t1 = be1 - rm1 * s1
    s2 = g2 / jnp.sqrt(rv2 + EPS)
    t2 = be2 - rm2 * s2

    w1f = W1 * s0[None, :] * s1[:, None]
    b1f = (t0 @ W1.T + b1) * s1 + t1
    w2f = W2 * s2[:, None]
    b2f = b2 * s2 + t2

    idx = x[:, 0].astype(jnp.int32)
    h = _sc_gather(emb, idx)
    out = _mlp(h, w1f.T, b1f[None, :], w2f.T, b2f[None, :],
               Wout.T, bout[None, :])
    return out


# speedup vs baseline: 1.0070x; 1.0070x over previous
import functools

import jax
import jax.numpy as jnp
from jax import lax
from jax.experimental import pallas as pl
from jax.experimental.pallas import tpu as pltpu
from jax.experimental.pallas import tpu_sc as plsc

B = 16384
V = 1000000
D = 64
H1 = 384
H2 = 192
NC_OUT = 173
EPS = 1e-5

_NUM_WORKERS = 32
_B_PER_W = B // _NUM_WORKERS


@functools.partial(
    pl.kernel,
    out_type=jax.ShapeDtypeStruct((B, 2 * D), jnp.float32),
    mesh=plsc.VectorSubcoreMesh(core_axis_name="c", subcore_axis_name="s"),
    scratch_types=[
        pltpu.VMEM((_B_PER_W,), jnp.int32),
        pltpu.VMEM((_B_PER_W, 2 * D), jnp.float32),
        pltpu.SemaphoreType.DMA,
    ],
    compiler_params=pltpu.CompilerParams(use_tc_tiling_on_sc=True),
)
def _sc_gather(table_hbm, idx2_hbm, out_hbm, idx_v, rows_v, sem):
    wid = lax.axis_index("s") * 2 + lax.axis_index("c")
    base = wid * _B_PER_W
    pltpu.sync_copy(idx2_hbm.at[pl.ds(base, _B_PER_W)], idx_v)
    pltpu.async_copy(table_hbm.at[idx_v], rows_v, sem).wait()
    pltpu.sync_copy(rows_v, out_hbm.at[pl.ds(base, _B_PER_W)])


def _mlp_t_body(h_ref, par_ref, w1_ref, b1_ref, w2_ref, b2_ref, wo_ref,
                bo_ref, out_ref):
    h128 = h_ref[...]
    par = par_ref[...]
    h = jnp.where(par > 0, h128[:, D:], h128[:, :D])
    z1 = lax.dot_general(w1_ref[...], h, (((1,), (1,)), ((), ())),
                         preferred_element_type=jnp.float32)
    z1 = z1 + b1_ref[...]
    h1 = 0.5 * z1 * (1.0 + lax.erf(z1 * 0.7071067811865476))
    z2 = jnp.dot(w2_ref[...], h1, preferred_element_type=jnp.float32)
    z2 = z2 + b2_ref[...]
    h2 = 0.5 * z2 * (1.0 + lax.erf(z2 * 0.7071067811865476))
    out = jnp.dot(wo_ref[...], h2, preferred_element_type=jnp.float32)
    out_ref[...] = out + bo_ref[...]


_BLK = 2048


def _mlp_t(h128, parity, w1f, b1f, w2f, b2f, wout, bout):
    grid = (B // _BLK,)
    return pl.pallas_call(
        _mlp_t_body,
        grid=grid,
        in_specs=[
            pl.BlockSpec((_BLK, 2 * D), lambda i: (i, 0)),
            pl.BlockSpec((_BLK, 1), lambda i: (i, 0)),
            pl.BlockSpec((H1, D), lambda i: (0, 0)),
            pl.BlockSpec((H1, 1), lambda i: (0, 0)),
            pl.BlockSpec((H2, H1), lambda i: (0, 0)),
            pl.BlockSpec((H2, 1), lambda i: (0, 0)),
            pl.BlockSpec((NC_OUT, H2), lambda i: (0, 0)),
            pl.BlockSpec((NC_OUT, 1), lambda i: (0, 0)),
        ],
        out_specs=pl.BlockSpec((NC_OUT, _BLK), lambda i: (0, i)),
        out_shape=jax.ShapeDtypeStruct((NC_OUT, B), jnp.float32),
    )(h128, parity, w1f, b1f, w2f, b2f, wout, bout)


def kernel(x, emb, g0, be0, rm0, rv0, W1, b1, g1, be1, rm1, rv1,
           W2, b2, g2, be2, rm2, rv2, Wout, bout):
    s0 = g0 / jnp.sqrt(rv0 + EPS)
    t0 = be0 - rm0 * s0
    s1 = g1 / jnp.sqrt(rv1 + EPS)
    t1 = be1 - rm1 * s1
    s2 = g2 / jnp.sqrt(rv2 + EPS)
    t2 = be2 - rm2 * s2

    w1f = W1 * s0[None, :] * s1[:, None]
    b1f = (t0 @ W1.T + b1) * s1 + t1
    w2f = W2 * s2[:, None]
    b2f = b2 * s2 + t2

    idx = x[:, 0].astype(jnp.int32)
    table2 = jnp.reshape(emb, (V // 2, 2 * D))
    h128 = _sc_gather(table2, idx >> 1)
    parity = (idx & 1)[:, None]
    out_t = _mlp_t(h128, parity, w1f, b1f[:, None], w2f, b2f[:, None],
                   Wout, bout[:, None])
    return out_t.T
